# baseline (speedup 1.0000x reference)
import functools

import jax
import jax.numpy as jnp
from jax import lax
from jax.experimental import pallas as pl
from jax.experimental.pallas import tpu as pltpu
from jax.experimental.pallas import tpu_sc as plsc

D = 256
K = 8192
TM = 512
TK = 1024
COMMIT = 0.25


def _argmin_body(x_ref, xn_ref, en_ref, emb_ref, idx_ref, loss_ref, acc_ref):
    i = pl.program_id(0)
    nsteps = pl.num_programs(0)
    x = x_ref[...]
    xn = xn_ref[...]

    def tile(t, carry):
        mn, ag = carry
        emb_t = emb_ref[pl.ds(t * TK, TK), :]
        en_t = en_ref[:, pl.ds(t * TK, TK)]
        mm = lax.dot_general(x, emb_t, (((1,), (1,)), ((), ())),
                             preferred_element_type=jnp.float32)
        d = (xn + en_t) - 2.0 * mm
        lm = jnp.min(d, axis=1, keepdims=True)
        cols = lax.broadcasted_iota(jnp.int32, d.shape, 1)
        la = jnp.min(jnp.where(d == lm, cols, K), axis=1, keepdims=True) + t * TK
        better = lm < mn
        return jnp.where(better, lm, mn), jnp.where(better, la, ag)

    mn0 = jnp.full((TM, 1), jnp.inf, jnp.float32)
    ag0 = jnp.zeros((TM, 1), jnp.int32)
    mn, ag = lax.fori_loop(0, K // TK, tile, (mn0, ag0))
    idx_ref[...] = ag

    @pl.when(i == 0)
    def _():
        acc_ref[0, 0] = 0.0

    acc_ref[0, 0] += jnp.sum(mn)

    @pl.when(i == nsteps - 1)
    def _():
        total = jnp.float32(nsteps * TM * D)
        loss_ref[0, 0] = (1.0 + COMMIT) * acc_ref[0, 0] / total


def _argmin_call(flat_x, xn, en, emb):
    m = flat_x.shape[0]
    grid = (m // TM,)
    return pl.pallas_call(
        _argmin_body,
        grid=grid,
        in_specs=[
            pl.BlockSpec((TM, D), lambda i: (i, 0)),
            pl.BlockSpec((TM, 1), lambda i: (i, 0)),
            pl.BlockSpec((1, K), lambda i: (0, 0)),
            pl.BlockSpec((K, D), lambda i: (0, 0)),
        ],
        out_specs=[
            pl.BlockSpec((TM, 1), lambda i: (i, 0)),
            pl.BlockSpec(memory_space=pltpu.SMEM),
        ],
        out_shape=[
            jax.ShapeDtypeStruct((m, 1), jnp.int32),
            jax.ShapeDtypeStruct((1, 1), jnp.float32),
        ],
        scratch_shapes=[pltpu.SMEM((1, 1), jnp.float32)],
    )(flat_x, xn, en, emb)


def _make_sc_gather(m):
    info = plsc.get_sparse_core_info()
    nw = info.num_cores * info.num_subcores
    nc = info.num_cores
    b_per_w = m // nw
    ch = 128
    nch = b_per_w // ch

    mesh = plsc.VectorSubcoreMesh(core_axis_name="c", subcore_axis_name="s")

    @functools.partial(
        pl.kernel,
        mesh=mesh,
        out_type=jax.ShapeDtypeStruct((m, D), jnp.float32),
        scratch_types=[
            pltpu.VMEM((nch, ch), jnp.int32),
            pltpu.VMEM((b_per_w, D), jnp.float32),
            pltpu.SemaphoreType.DMA,
        ],
    )
    def gather_k(table_hbm, idx_hbm, out_hbm, idx_v, rows_v, sem):
        wid = lax.axis_index("s") * nc + lax.axis_index("c")
        base = wid * b_per_w
        pltpu.sync_copy(idx_hbm.at[pl.ds(wid * nch, nch)], idx_v)
        copies = [
            pltpu.async_copy(table_hbm.at[idx_v.at[j]],
                             rows_v.at[pl.ds(j * ch, ch)], sem)
            for j in range(nch)
        ]
        for cp in copies:
            cp.wait()
        pltpu.sync_copy(rows_v, out_hbm.at[pl.ds(base, b_per_w)])

    return gather_k, nch, ch


def kernel(x, embeddings_weight):
    b, c, h, w = x.shape
    m = b * h * w
    xp = jnp.transpose(x, (0, 2, 3, 1))
    flat_x = xp.reshape(m, c)
    xn = jnp.sum(flat_x ** 2, axis=1, keepdims=True)
    en = jnp.sum(embeddings_weight ** 2, axis=1)[None, :]

    idx2d, loss2d = _argmin_call(flat_x, xn, en, embeddings_weight)

    gather_k, nch, ch = _make_sc_gather(m)
    idx_chunks = idx2d.reshape(m // ch, ch)
    q = gather_k(embeddings_weight, idx_chunks)

    quantized_out = q.reshape(b, h, w, c).transpose(0, 3, 1, 2)
    return quantized_out, loss2d[0, 0]


# speedup vs baseline: 1.6703x; 1.6703x over previous
import functools

import jax
import jax.numpy as jnp
from jax import lax
from jax.experimental import pallas as pl
from jax.experimental.pallas import tpu as pltpu
from jax.experimental.pallas import tpu_sc as plsc

D = 256
K = 8192
TM = 512
TK = 1024
COMMIT = 0.25


def _argmin_body(x2_ref, xn_ref, en_ref, emb_ref, idx_ref, loss_ref, acc_ref):
    i = pl.program_id(0)
    nsteps = pl.num_programs(0)
    x2 = x2_ref[...]
    xn = xn_ref[...]

    lanes = 128
    lane_iota = lax.broadcasted_iota(jnp.int32, (TM, lanes), 1)
    rmin = jnp.full((TM, lanes), jnp.inf, jnp.float32)
    rarg = jnp.zeros((TM, lanes), jnp.int32)

    for t in range(K // TK):
        emb_t = emb_ref[pl.ds(t * TK, TK), :]
        mm = lax.dot_general(x2, emb_t, (((1,), (1,)), ((), ())),
                             preferred_element_type=jnp.float32)
        for g in range(TK // lanes):
            en_g = en_ref[:, pl.ds(t * TK + g * lanes, lanes)]
            d = (xn + en_g) + mm[:, g * lanes:(g + 1) * lanes]
            better = d < rmin
            rmin = jnp.where(better, d, rmin)
            rarg = jnp.where(better, lane_iota + (t * TK + g * lanes), rarg)

    mn = jnp.min(rmin, axis=1, keepdims=True)
    ag = jnp.min(jnp.where(rmin == mn, rarg, K), axis=1, keepdims=True)
    idx_ref[...] = ag

    @pl.when(i == 0)
    def _():
        acc_ref[0, 0] = 0.0

    acc_ref[0, 0] += jnp.sum(mn)

    @pl.when(i == nsteps - 1)
    def _():
        total = jnp.float32(nsteps * TM * D)
        loss_ref[0, 0] = (1.0 + COMMIT) * acc_ref[0, 0] / total


def _argmin_call(x2, xn, en, emb):
    m = x2.shape[0]
    grid = (m // TM,)
    return pl.pallas_call(
        _argmin_body,
        grid=grid,
        in_specs=[
            pl.BlockSpec((TM, D), lambda i: (i, 0)),
            pl.BlockSpec((TM, 1), lambda i: (i, 0)),
            pl.BlockSpec((1, K), lambda i: (0, 0)),
            pl.BlockSpec((K, D), lambda i: (0, 0)),
        ],
        out_specs=[
            pl.BlockSpec((TM, 1), lambda i: (i, 0)),
            pl.BlockSpec(memory_space=pltpu.SMEM),
        ],
        out_shape=[
            jax.ShapeDtypeStruct((m, 1), jnp.int32),
            jax.ShapeDtypeStruct((1, 1), jnp.float32),
        ],
        scratch_shapes=[pltpu.SMEM((1, 1), jnp.float32)],
    )(x2, xn, en, emb)


def _make_sc_gather(m):
    info = plsc.get_sparse_core_info()
    nw = info.num_cores * info.num_subcores
    nc = info.num_cores
    b_per_w = m // nw
    ch = 128
    nch = b_per_w // ch

    mesh = plsc.VectorSubcoreMesh(core_axis_name="c", subcore_axis_name="s")

    @functools.partial(
        pl.kernel,
        mesh=mesh,
        out_type=jax.ShapeDtypeStruct((m, D), jnp.float32),
        scratch_types=[
            pltpu.VMEM((nch, ch), jnp.int32),
            pltpu.VMEM((b_per_w, D), jnp.float32),
            pltpu.SemaphoreType.DMA,
        ],
    )
    def gather_k(table_hbm, idx_hbm, out_hbm, idx_v, rows_v, sem):
        wid = lax.axis_index("s") * nc + lax.axis_index("c")
        base = wid * b_per_w
        pltpu.sync_copy(idx_hbm.at[pl.ds(wid * nch, nch)], idx_v)
        copies = [
            pltpu.async_copy(table_hbm.at[idx_v.at[j]],
                             rows_v.at[pl.ds(j * ch, ch)], sem)
            for j in range(nch)
        ]
        for cp in copies:
            cp.wait()
        pltpu.sync_copy(rows_v, out_hbm.at[pl.ds(base, b_per_w)])

    return gather_k, nch, ch


def kernel(x, embeddings_weight):
    b, c, h, w = x.shape
    m = b * h * w
    xp = jnp.transpose(x, (0, 2, 3, 1))
    flat_x = xp.reshape(m, c)
    xn = jnp.sum(flat_x ** 2, axis=1, keepdims=True)
    en = jnp.sum(embeddings_weight ** 2, axis=1)[None, :]
    x2 = flat_x * -2.0

    idx2d, loss2d = _argmin_call(x2, xn, en, embeddings_weight)

    gather_k, nch, ch = _make_sc_gather(m)
    idx_chunks = idx2d.reshape(m // ch, ch)
    q = gather_k(embeddings_weight, idx_chunks)

    quantized_out = q.reshape(b, h, w, c).transpose(0, 3, 1, 2)
    return quantized_out, loss2d[0, 0]
